# baseline (speedup 1.0000x reference)
import functools

import jax
import jax.numpy as jnp
from jax import lax
from jax.experimental import pallas as pl
from jax.experimental.pallas import tpu as pltpu
from jax.experimental.pallas import tpu_sc as plsc

B = 16384
D = 128
NC = 2
NS = 16
NW = NC * NS
B_PER_W = B // NW


@jax.jit
def kernel(indices, table):
    mesh = plsc.VectorSubcoreMesh(core_axis_name="c", subcore_axis_name="s")

    @functools.partial(
        pl.kernel,
        mesh=mesh,
        out_type=jax.ShapeDtypeStruct((B, D), jnp.float32),
        scratch_types=[
            pltpu.VMEM((B_PER_W,), jnp.int32),
            pltpu.VMEM((B_PER_W, D), jnp.float32),
            pltpu.SemaphoreType.DMA,
        ],
    )
    def gather_kernel(table_hbm, idx_hbm, out_hbm, idx_v, rows_v, sem):
        wid = lax.axis_index("s") * NC + lax.axis_index("c")
        base = wid * B_PER_W
        pltpu.sync_copy(idx_hbm.at[pl.ds(base, B_PER_W)], idx_v)
        pltpu.async_copy(table_hbm.at[idx_v], rows_v, sem).wait()
        pltpu.sync_copy(rows_v, out_hbm.at[pl.ds(base, B_PER_W)])

    return gather_kernel(table, indices.astype(jnp.int32))


# speedup vs baseline: 3.6773x; 3.6773x over previous
import functools

import jax
import jax.numpy as jnp
from jax import lax
from jax.experimental import pallas as pl
from jax.experimental.pallas import tpu as pltpu
from jax.experimental.pallas import tpu_sc as plsc

B = 16384
D = 128
V = 10
NC = 2
NS = 16
NW = NC * NS
B_PER_W = B // NW


@jax.jit
def kernel(indices, table):
    mesh = plsc.VectorSubcoreMesh(core_axis_name="c", subcore_axis_name="s")

    @functools.partial(
        pl.kernel,
        mesh=mesh,
        out_type=jax.ShapeDtypeStruct((B, D), jnp.float32),
        scratch_types=[
            pltpu.VMEM((B_PER_W,), jnp.int32),
            pltpu.VMEM_SHARED((V, D), jnp.float32),
            pltpu.VMEM((B_PER_W, D), jnp.float32),
            pltpu.SemaphoreType.DMA,
        ],
    )
    def lookup_kernel(table_hbm, idx_hbm, out_hbm, idx_v, tbl_sh, rows_v, sem):
        sid = lax.axis_index("s")
        wid = sid * NC + lax.axis_index("c")
        base = wid * B_PER_W

        @pl.when(sid == 0)
        def _():
            pltpu.sync_copy(table_hbm, tbl_sh)

        pltpu.sync_copy(idx_hbm.at[pl.ds(base, B_PER_W)], idx_v)
        plsc.subcore_barrier()
        pltpu.async_copy(tbl_sh.at[idx_v], rows_v, sem).wait()
        pltpu.sync_copy(rows_v, out_hbm.at[pl.ds(base, B_PER_W)])

    return lookup_kernel(table, indices.astype(jnp.int32))


# speedup vs baseline: 3.8069x; 1.0353x over previous
import functools

import jax
import jax.numpy as jnp
from jax import lax
from jax.experimental import pallas as pl
from jax.experimental.pallas import tpu as pltpu
from jax.experimental.pallas import tpu_sc as plsc

B = 16384
D = 128
V = 10
NC = 2
NS = 16
NW = NC * NS
B_PER_W = B // NW
NCHUNK = 4
CH = B_PER_W // NCHUNK


@jax.jit
def kernel(indices, table):
    mesh = plsc.VectorSubcoreMesh(core_axis_name="c", subcore_axis_name="s")

    @functools.partial(
        pl.kernel,
        mesh=mesh,
        out_type=jax.ShapeDtypeStruct((B, D), jnp.float32),
        scratch_types=[
            pltpu.VMEM((B_PER_W,), jnp.int32),
            pltpu.VMEM_SHARED((V, D), jnp.float32),
            pltpu.VMEM((B_PER_W, D), jnp.float32),
            pltpu.SemaphoreType.DMA,
            pltpu.SemaphoreType.DMA,
        ],
    )
    def lookup_kernel(table_hbm, idx_hbm, out_hbm, idx_v, tbl_sh, rows_v,
                      gsem, wsem):
        sid = lax.axis_index("s")
        wid = sid * NC + lax.axis_index("c")
        base = wid * B_PER_W

        @pl.when(sid == 0)
        def _():
            pltpu.sync_copy(table_hbm, tbl_sh)

        pltpu.sync_copy(idx_hbm.at[pl.ds(base, B_PER_W)], idx_v)
        plsc.subcore_barrier()

        gathers = [
            pltpu.async_copy(
                tbl_sh.at[idx_v.at[pl.ds(k * CH, CH)]],
                rows_v.at[pl.ds(k * CH, CH)],
                gsem,
            )
            for k in range(NCHUNK)
        ]
        writes = []
        for k in range(NCHUNK):
            gathers[k].wait()
            writes.append(
                pltpu.async_copy(
                    rows_v.at[pl.ds(k * CH, CH)],
                    out_hbm.at[pl.ds(base + k * CH, CH)],
                    wsem,
                )
            )
        for w in writes:
            w.wait()

    return lookup_kernel(table, indices.astype(jnp.int32))


# speedup vs baseline: 3.8853x; 1.0206x over previous
import functools

import jax
import jax.numpy as jnp
from jax import lax
from jax.experimental import pallas as pl
from jax.experimental.pallas import tpu as pltpu
from jax.experimental.pallas import tpu_sc as plsc

B = 16384
D = 128
V = 10
NC = 2
NS = 16
NW = NC * NS
B_PER_W = B // NW
NCHUNK = 8
CH = B_PER_W // NCHUNK


@jax.jit
def kernel(indices, table):
    mesh = plsc.VectorSubcoreMesh(core_axis_name="c", subcore_axis_name="s")

    @functools.partial(
        pl.kernel,
        mesh=mesh,
        out_type=jax.ShapeDtypeStruct((B, D), jnp.float32),
        scratch_types=[
            pltpu.VMEM((B_PER_W,), jnp.int32),
            pltpu.VMEM_SHARED((V, D), jnp.float32),
            pltpu.VMEM((B_PER_W, D), jnp.float32),
            pltpu.SemaphoreType.DMA,
            pltpu.SemaphoreType.DMA,
        ],
    )
    def lookup_kernel(table_hbm, idx_hbm, out_hbm, idx_v, tbl_sh, rows_v,
                      gsem, wsem):
        sid = lax.axis_index("s")
        wid = sid * NC + lax.axis_index("c")
        base = wid * B_PER_W

        idx_cp = pltpu.async_copy(idx_hbm.at[pl.ds(base, B_PER_W)], idx_v,
                                  wsem)

        @pl.when(sid == 0)
        def _():
            pltpu.sync_copy(table_hbm, tbl_sh)

        idx_cp.wait()
        plsc.subcore_barrier()

        gathers = [
            pltpu.async_copy(
                tbl_sh.at[idx_v.at[pl.ds(k * CH, CH)]],
                rows_v.at[pl.ds(k * CH, CH)],
                gsem,
            )
            for k in range(NCHUNK)
        ]
        writes = []
        for k in range(NCHUNK):
            gathers[k].wait()
            writes.append(
                pltpu.async_copy(
                    rows_v.at[pl.ds(k * CH, CH)],
                    out_hbm.at[pl.ds(base + k * CH, CH)],
                    wsem,
                )
            )
        for w in writes:
            w.wait()

    return lookup_kernel(table, indices.astype(jnp.int32))
